# Initial kernel scaffold; baseline (speedup 1.0000x reference)
import jax
import jax.numpy as jnp
from jax.experimental import pallas as pl


def kernel(x, uv):
    raise NotImplementedError("write your pallas kernel here")


# speedup vs baseline: 55.6116x; 55.6116x over previous
import functools
import math

import jax
import jax.numpy as jnp
import numpy as np
from jax import lax
from jax.experimental import pallas as pl
from jax.experimental.pallas import tpu as pltpu
from jax.experimental.pallas import tpu_sc as plsc

H = 256
W = 256
KH = 512
KW = 512
J = 7
ALPHA = 2.34 * J
N = 100000
B = 4
NSHIFT = (H // 2, W // 2)

NCHUNK = 16
NWORK = 32
GPW = 25
GSIZE = 128
NP = NWORK * GPW * GSIZE

I0_ALPHA = float(np.i0(ALPHA))
KSCALE0 = KH / (2.0 * math.pi)
KSCALE1 = KW / (2.0 * math.pi)


def _np_kb_ft(om, numpoints, alpha):
    z2 = (np.pi * numpoints * om) ** 2 - alpha**2
    z = np.sqrt(z2.astype(np.complex128))
    return numpoints * np.real(np.sin(z) / z) / np.i0(alpha)


def _np_scaling_coefs(num, grid, numpoints, alpha):
    pos = (np.arange(num, dtype=np.float64) - (num - 1) / 2.0) / grid
    return 1.0 / _np_kb_ft(pos, numpoints, alpha)


_SN = np.outer(
    _np_scaling_coefs(H, KH, J, ALPHA), _np_scaling_coefs(W, KW, J, ALPHA)
).astype(np.float32)

_k = np.arange(KH, dtype=np.float64)[:, None]
_r = np.arange(H, dtype=np.float64)[None, :]
_ANG = 2.0 * np.pi * _k * _r / KH
_C1 = (np.cos(_ANG) / np.sqrt(KH)).astype(np.float32)
_S1 = (np.sin(_ANG) / np.sqrt(KH)).astype(np.float32)


def _dot(a, b, da, db):
    return lax.dot_general(
        a,
        b,
        (((da,), (db,)), ((), ())),
        preferred_element_type=jnp.float32,
        precision=lax.Precision.HIGHEST,
    )


def _dft_body(x_ref, sn_ref, c_ref, s_ref, out_ref):
    c = c_ref[...]
    s = s_ref[...]
    sn = sn_ref[...]
    for b in range(B):
        xs = x_ref[b] * sn
        ac = _dot(c, xs, 1, 0)
        as_ = _dot(s, xs, 1, 0)
        out_ref[b] = _dot(ac, c, 1, 1) - _dot(as_, s, 1, 1)
        out_ref[B + b] = -(_dot(ac, s, 1, 1) + _dot(as_, c, 1, 1))


def _i0(x):
    t = (x * (1.0 / 3.75)) ** 2
    small = 1.0 + t * (
        3.5156229
        + t
        * (3.0899424 + t * (1.2067492 + t * (0.2659732 + t * (0.0360768 + t * 0.0045813))))
    )
    xi = 1.0 / jnp.maximum(x, 3.75)
    u = 3.75 * xi
    big = 0.39894228 + u * (
        0.01328592
        + u
        * (
            0.00225319
            + u
            * (
                -0.00157565
                + u
                * (
                    0.00916281
                    + u * (-0.02057706 + u * (0.02635537 + u * (-0.01647633 + u * 0.00392377)))
                )
            )
        )
    )
    big = big * jnp.exp(x) * jnp.sqrt(xi)
    return jnp.where(x < 3.75, small, big)


def _kb_weight(u):
    arg = 1.0 - (2.0 * u / J) ** 2
    pos = arg > 0
    sq = jnp.sqrt(jnp.where(pos, arg, 1.0))
    w = _i0(ALPHA * jnp.where(pos, sq, 0.0)) * (1.0 / I0_ALPHA)
    return jnp.where(pos, w, 0.0)


def _prep_body(
    u0_ref, u1_ref, w0_ref, w1_ref, rb_ref, hi_ref, hi2_ref, off_ref, pc_ref, ps_ref
):
    u0 = u0_ref[...]
    u1 = u1_ref[...]
    tm0 = u0 * KSCALE0
    tm1 = u1 * KSCALE1
    r0 = jnp.rint(tm0)
    r1 = jnp.rint(tm1)
    f0 = tm0 - r0
    f1 = tm1 - r1
    for j in range(J):
        off = float(j) - (J - 1) / 2.0
        w0_ref[j] = _kb_weight(f0 - off)
        w1_ref[j] = _kb_weight(f1 - off)
        k0 = (r0 + off).astype(jnp.int32)
        rb_ref[j] = jnp.mod(k0, KH) * 32
    c0 = jnp.mod((r1 - 3.0).astype(jnp.int32), KW)
    hi = lax.shift_right_logical(c0, 4)
    hi_ref[...] = hi
    hi2_ref[...] = jnp.bitwise_and(hi + 1, 31)
    off_ref[...] = jnp.bitwise_and(c0, 15) * 8
    phase = u0 * float(NSHIFT[0]) + u1 * float(NSHIFT[1])
    pc_ref[...] = jnp.cos(phase)
    ps_ref[...] = jnp.sin(phase)


def _interp_body(
    table,
    rb,
    hi,
    hi2,
    off0,
    w0,
    w1,
    pc,
    ps,
    out,
    rbg,
    w0g,
    w1g,
    hig,
    hi2g,
    offg,
    pcg,
    psg,
    idx_s,
    buf_s,
    outg,
    sem,
):
    wid = lax.axis_index("s") * 2 + lax.axis_index("c")
    iota16 = lax.iota(jnp.int32, 16)
    zero16 = jnp.zeros((16,), jnp.float32)

    def group(g, carry):
        gbase = (wid * GPW + g) * GSIZE
        pltpu.sync_copy(rb.at[:, pl.ds(gbase, GSIZE)], rbg)
        pltpu.sync_copy(w0.at[:, pl.ds(gbase, GSIZE)], w0g)
        pltpu.sync_copy(w1.at[:, pl.ds(gbase, GSIZE)], w1g)
        pltpu.sync_copy(hi.at[pl.ds(gbase, GSIZE)], hig)
        pltpu.sync_copy(hi2.at[pl.ds(gbase, GSIZE)], hi2g)
        pltpu.sync_copy(off0.at[pl.ds(gbase, GSIZE)], offg)
        pltpu.sync_copy(pc.at[pl.ds(gbase, GSIZE)], pcg)
        pltpu.sync_copy(ps.at[pl.ds(gbase, GSIZE)], psg)

        def chunk(k, carry2):
            off = k * NCHUNK
            hiv = hig[pl.ds(off, NCHUNK)]
            hi2v = hi2g[pl.ds(off, NCHUNK)]
            offv = offg[pl.ds(off, NCHUNK)]
            ie = iota16 * 2
            for j0 in range(J):
                rbv = rbg[j0, pl.ds(off, NCHUNK)]
                plsc.store_scatter(idx_s.at[j0], [ie], rbv + hiv)
                plsc.store_scatter(idx_s.at[j0], [ie + 1], rbv + hi2v)
            for j0 in range(J):
                pltpu.async_copy(
                    table.at[idx_s.at[j0]], buf_s.at[pl.ds(j0 * 32, 32)], sem
                ).wait()
            w0l = [w0g[j, pl.ds(off, NCHUNK)] for j in range(J)]
            w1l = [w1g[j, pl.ds(off, NCHUNK)] for j in range(J)]
            sjl = []
            cjl = []
            for j1 in range(J):
                tj = offv + (j1 * 8)
                sjl.append(lax.shift_right_logical(tj, 7))
                cjl.append(jnp.bitwise_and(tj, 127))
            acc = [zero16 for _ in range(8)]
            for j0 in range(J):
                rowb = ie + (j0 * 32)
                for j1 in range(J):
                    wt = w0l[j0] * w1l[j1]
                    rowj = rowb + sjl[j1]
                    cj = cjl[j1]
                    for u in range(8):
                        v = plsc.load_gather(buf_s, [rowj, cj + u])
                        acc[u] = acc[u] + wt * v
            pcl = pcg[pl.ds(off, NCHUNK)]
            psl = psg[pl.ds(off, NCHUNK)]
            for c in range(4):
                outg[c, pl.ds(off, NCHUNK)] = acc[c] * pcl - acc[4 + c] * psl
                outg[4 + c, pl.ds(off, NCHUNK)] = acc[c] * psl + acc[4 + c] * pcl
            return carry2

        lax.fori_loop(0, GSIZE // NCHUNK, chunk, 0)
        pltpu.sync_copy(outg, out.at[:, pl.ds(gbase, GSIZE)])
        return carry

    lax.fori_loop(0, GPW, group, 0)


def kernel(x, uv):
    xb = x.reshape(B, H, W)
    up = jnp.pad(uv, ((0, 0), (0, NP - N)))
    u0 = up[0].reshape(NP // 128, 128)
    u1 = up[1].reshape(NP // 128, 128)

    sn = jnp.asarray(_SN)
    c1 = jnp.asarray(_C1)
    s1 = jnp.asarray(_S1)

    stacked = pl.pallas_call(
        _dft_body,
        out_shape=jax.ShapeDtypeStruct((2 * B, KH, KW), jnp.float32),
    )(xb, sn, c1, s1)
    table = stacked.transpose(1, 2, 0).reshape(KH * 32, 16 * 2 * B)

    pshape = jax.ShapeDtypeStruct((J, NP // 128, 128), jnp.float32)
    ishape = jax.ShapeDtypeStruct((J, NP // 128, 128), jnp.int32)
    vshape = jax.ShapeDtypeStruct((NP // 128, 128), jnp.float32)
    sshape = jax.ShapeDtypeStruct((NP // 128, 128), jnp.int32)
    w0, w1, rb, hi, hi2, off0, pc, ps = pl.pallas_call(
        _prep_body,
        out_shape=(pshape, pshape, ishape, sshape, sshape, sshape, vshape, vshape),
    )(u0, u1)

    mesh = plsc.VectorSubcoreMesh(
        core_axis_name="c", subcore_axis_name="s", num_cores=2, num_subcores=16
    )
    interp = pl.kernel(
        _interp_body,
        out_type=jax.ShapeDtypeStruct((8, NP), jnp.float32),
        mesh=mesh,
        scratch_types=[
            pltpu.VMEM((J, GSIZE), jnp.int32),
            pltpu.VMEM((J, GSIZE), jnp.float32),
            pltpu.VMEM((J, GSIZE), jnp.float32),
            pltpu.VMEM((GSIZE,), jnp.int32),
            pltpu.VMEM((GSIZE,), jnp.int32),
            pltpu.VMEM((GSIZE,), jnp.int32),
            pltpu.VMEM((GSIZE,), jnp.float32),
            pltpu.VMEM((GSIZE,), jnp.float32),
            pltpu.VMEM((J, 32), jnp.int32),
            pltpu.VMEM((J * 32, 128), jnp.float32),
            pltpu.VMEM((8, GSIZE), jnp.float32),
            pltpu.SemaphoreType.DMA,
        ],
        compiler_params=pltpu.CompilerParams(needs_layout_passes=False),
    )
    res = interp(
        table,
        rb.reshape(J, NP),
        hi.reshape(NP),
        hi2.reshape(NP),
        off0.reshape(NP),
        w0.reshape(J, NP),
        w1.reshape(J, NP),
        pc.reshape(NP),
        ps.reshape(NP),
    )
    kdata = res[0:B, :N] + 1j * res[B : 2 * B, :N]
    return kdata.astype(jnp.complex64)[:, None, :]


# speedup vs baseline: 107.6072x; 1.9350x over previous
import functools
import math

import jax
import jax.numpy as jnp
import numpy as np
from jax import lax
from jax.experimental import pallas as pl
from jax.experimental.pallas import tpu as pltpu
from jax.experimental.pallas import tpu_sc as plsc

H = 256
W = 256
KH = 512
KW = 512
J = 7
ALPHA = 2.34 * J
N = 100000
B = 4
NSHIFT = (H // 2, W // 2)

NCHUNK = 16
NWORK = 32
GPW = 25
GSIZE = 128
NP = NWORK * GPW * GSIZE

I0_ALPHA = float(np.i0(ALPHA))
KSCALE0 = KH / (2.0 * math.pi)
KSCALE1 = KW / (2.0 * math.pi)


def _np_kb_ft(om, numpoints, alpha):
    z2 = (np.pi * numpoints * om) ** 2 - alpha**2
    z = np.sqrt(z2.astype(np.complex128))
    return numpoints * np.real(np.sin(z) / z) / np.i0(alpha)


def _np_scaling_coefs(num, grid, numpoints, alpha):
    pos = (np.arange(num, dtype=np.float64) - (num - 1) / 2.0) / grid
    return 1.0 / _np_kb_ft(pos, numpoints, alpha)


_SN = np.outer(
    _np_scaling_coefs(H, KH, J, ALPHA), _np_scaling_coefs(W, KW, J, ALPHA)
).astype(np.float32)

_k = np.arange(KH, dtype=np.float64)[:, None]
_r = np.arange(H, dtype=np.float64)[None, :]
_ANG = 2.0 * np.pi * _k * _r / KH
_C1 = (np.cos(_ANG) / np.sqrt(KH)).astype(np.float32)
_S1 = (np.sin(_ANG) / np.sqrt(KH)).astype(np.float32)


def _dot(a, b, da, db):
    return lax.dot_general(
        a,
        b,
        (((da,), (db,)), ((), ())),
        preferred_element_type=jnp.float32,
        precision=lax.Precision.HIGHEST,
    )


def _dft_body(x_ref, sn_ref, c_ref, s_ref, out_ref):
    c = c_ref[...]
    s = s_ref[...]
    sn = sn_ref[...]
    for b in range(B):
        xs = x_ref[b] * sn
        ac = _dot(c, xs, 1, 0)
        as_ = _dot(s, xs, 1, 0)
        out_ref[b] = _dot(ac, c, 1, 1) - _dot(as_, s, 1, 1)
        out_ref[B + b] = -(_dot(ac, s, 1, 1) + _dot(as_, c, 1, 1))


def _i0(x):
    t = (x * (1.0 / 3.75)) ** 2
    small = 1.0 + t * (
        3.5156229
        + t
        * (3.0899424 + t * (1.2067492 + t * (0.2659732 + t * (0.0360768 + t * 0.0045813))))
    )
    xi = 1.0 / jnp.maximum(x, 3.75)
    u = 3.75 * xi
    big = 0.39894228 + u * (
        0.01328592
        + u
        * (
            0.00225319
            + u
            * (
                -0.00157565
                + u
                * (
                    0.00916281
                    + u * (-0.02057706 + u * (0.02635537 + u * (-0.01647633 + u * 0.00392377)))
                )
            )
        )
    )
    big = big * jnp.exp(x) * jnp.sqrt(xi)
    return jnp.where(x < 3.75, small, big)


def _kb_weight(u):
    arg = 1.0 - (2.0 * u / J) ** 2
    pos = arg > 0
    sq = jnp.sqrt(jnp.where(pos, arg, 1.0))
    w = _i0(ALPHA * jnp.where(pos, sq, 0.0)) * (1.0 / I0_ALPHA)
    return jnp.where(pos, w, 0.0)


def _prep_body(
    u0_ref, u1_ref, w0_ref, w1_ref, rb_ref, hi_ref, hi2_ref, off_ref, pc_ref, ps_ref
):
    u0 = u0_ref[...]
    u1 = u1_ref[...]
    tm0 = u0 * KSCALE0
    tm1 = u1 * KSCALE1
    r0 = jnp.rint(tm0)
    r1 = jnp.rint(tm1)
    f0 = tm0 - r0
    f1 = tm1 - r1
    for j in range(J):
        off = float(j) - (J - 1) / 2.0
        w0_ref[j] = _kb_weight(f0 - off)
        w1_ref[j] = _kb_weight(f1 - off)
        k0 = (r0 + off).astype(jnp.int32)
        rb_ref[j] = jnp.mod(k0, KH) * 32
    c0 = jnp.mod((r1 - 3.0).astype(jnp.int32), KW)
    hi = lax.shift_right_logical(c0, 4)
    hi_ref[...] = hi
    hi2_ref[...] = jnp.bitwise_and(hi + 1, 31)
    off_ref[...] = jnp.bitwise_and(c0, 15) * 8
    phase = u0 * float(NSHIFT[0]) + u1 * float(NSHIFT[1])
    pc_ref[...] = jnp.cos(phase)
    ps_ref[...] = jnp.sin(phase)


def _interp_body(
    table,
    rb,
    hi,
    hi2,
    off0,
    w0,
    w1,
    pc,
    ps,
    out,
    rbg,
    w0g,
    w1g,
    hig,
    hi2g,
    offg,
    pcg,
    psg,
    idx_s,
    buf_s,
    outg,
    sem,
):
    wid = lax.axis_index("s") * 2 + lax.axis_index("c")
    iota16 = lax.iota(jnp.int32, 16)
    zero16 = jnp.zeros((16,), jnp.float32)

    def group(g, carry):
        gbase = (wid * GPW + g) * GSIZE
        pltpu.sync_copy(rb.at[:, pl.ds(gbase, GSIZE)], rbg)
        pltpu.sync_copy(w0.at[:, pl.ds(gbase, GSIZE)], w0g)
        pltpu.sync_copy(w1.at[:, pl.ds(gbase, GSIZE)], w1g)
        pltpu.sync_copy(hi.at[pl.ds(gbase, GSIZE)], hig)
        pltpu.sync_copy(hi2.at[pl.ds(gbase, GSIZE)], hi2g)
        pltpu.sync_copy(off0.at[pl.ds(gbase, GSIZE)], offg)
        pltpu.sync_copy(pc.at[pl.ds(gbase, GSIZE)], pcg)
        pltpu.sync_copy(ps.at[pl.ds(gbase, GSIZE)], psg)

        def chunk(k, carry2):
            off = k * NCHUNK
            hiv = hig[pl.ds(off, NCHUNK)]
            hi2v = hi2g[pl.ds(off, NCHUNK)]
            offv = offg[pl.ds(off, NCHUNK)]
            ie = iota16 * 2
            for j0 in range(J):
                rbv = rbg[j0, pl.ds(off, NCHUNK)]
                plsc.store_scatter(idx_s.at[j0], [ie], rbv + hiv)
                plsc.store_scatter(idx_s.at[j0], [ie + 1], rbv + hi2v)
            copies = [
                pltpu.async_copy(
                    table.at[idx_s.at[j0]], buf_s.at[pl.ds(j0 * 32, 32)], sem
                )
                for j0 in range(J)
            ]
            for cp in copies:
                cp.wait()
            w0l = [w0g[j, pl.ds(off, NCHUNK)] for j in range(J)]
            w1l = [w1g[j, pl.ds(off, NCHUNK)] for j in range(J)]
            sjl = []
            cjl = []
            for j1 in range(J):
                tj = offv + (j1 * 8)
                sjl.append(lax.shift_right_logical(tj, 7))
                cjl.append(jnp.bitwise_and(tj, 127))
            acc = [zero16 for _ in range(8)]
            for j0 in range(J):
                rowb = ie + (j0 * 32)
                for j1 in range(J):
                    wt = w0l[j0] * w1l[j1]
                    rowj = rowb + sjl[j1]
                    cj = cjl[j1]
                    for u in range(8):
                        v = plsc.load_gather(buf_s, [rowj, cj + u])
                        acc[u] = acc[u] + wt * v
            pcl = pcg[pl.ds(off, NCHUNK)]
            psl = psg[pl.ds(off, NCHUNK)]
            for c in range(4):
                outg[c, pl.ds(off, NCHUNK)] = acc[c] * pcl - acc[4 + c] * psl
                outg[4 + c, pl.ds(off, NCHUNK)] = acc[c] * psl + acc[4 + c] * pcl
            return carry2

        lax.fori_loop(0, GSIZE // NCHUNK, chunk, 0)
        pltpu.sync_copy(outg, out.at[:, pl.ds(gbase, GSIZE)])
        return carry

    lax.fori_loop(0, GPW, group, 0)


def kernel(x, uv):
    xb = x.reshape(B, H, W)
    up = jnp.pad(uv, ((0, 0), (0, NP - N)))
    u0 = up[0].reshape(NP // 128, 128)
    u1 = up[1].reshape(NP // 128, 128)

    sn = jnp.asarray(_SN)
    c1 = jnp.asarray(_C1)
    s1 = jnp.asarray(_S1)

    stacked = pl.pallas_call(
        _dft_body,
        out_shape=jax.ShapeDtypeStruct((2 * B, KH, KW), jnp.float32),
    )(xb, sn, c1, s1)
    table = stacked.transpose(1, 2, 0).reshape(KH * 32, 16 * 2 * B)

    pshape = jax.ShapeDtypeStruct((J, NP // 128, 128), jnp.float32)
    ishape = jax.ShapeDtypeStruct((J, NP // 128, 128), jnp.int32)
    vshape = jax.ShapeDtypeStruct((NP // 128, 128), jnp.float32)
    sshape = jax.ShapeDtypeStruct((NP // 128, 128), jnp.int32)
    w0, w1, rb, hi, hi2, off0, pc, ps = pl.pallas_call(
        _prep_body,
        out_shape=(pshape, pshape, ishape, sshape, sshape, sshape, vshape, vshape),
    )(u0, u1)

    mesh = plsc.VectorSubcoreMesh(
        core_axis_name="c", subcore_axis_name="s", num_cores=2, num_subcores=16
    )
    interp = pl.kernel(
        _interp_body,
        out_type=jax.ShapeDtypeStruct((8, NP), jnp.float32),
        mesh=mesh,
        scratch_types=[
            pltpu.VMEM((J, GSIZE), jnp.int32),
            pltpu.VMEM((J, GSIZE), jnp.float32),
            pltpu.VMEM((J, GSIZE), jnp.float32),
            pltpu.VMEM((GSIZE,), jnp.int32),
            pltpu.VMEM((GSIZE,), jnp.int32),
            pltpu.VMEM((GSIZE,), jnp.int32),
            pltpu.VMEM((GSIZE,), jnp.float32),
            pltpu.VMEM((GSIZE,), jnp.float32),
            pltpu.VMEM((J, 32), jnp.int32),
            pltpu.VMEM((J * 32, 128), jnp.float32),
            pltpu.VMEM((8, GSIZE), jnp.float32),
            pltpu.SemaphoreType.DMA,
        ],
        compiler_params=pltpu.CompilerParams(needs_layout_passes=False),
    )
    res = interp(
        table,
        rb.reshape(J, NP),
        hi.reshape(NP),
        hi2.reshape(NP),
        off0.reshape(NP),
        w0.reshape(J, NP),
        w1.reshape(J, NP),
        pc.reshape(NP),
        ps.reshape(NP),
    )
    kdata = res[0:B, :N] + 1j * res[B : 2 * B, :N]
    return kdata.astype(jnp.complex64)[:, None, :]


# speedup vs baseline: 138.4995x; 1.2871x over previous
import functools
import math

import jax
import jax.numpy as jnp
import numpy as np
from jax import lax
from jax.experimental import pallas as pl
from jax.experimental.pallas import tpu as pltpu
from jax.experimental.pallas import tpu_sc as plsc

H = 256
W = 256
KH = 512
KW = 512
J = 7
ALPHA = 2.34 * J
N = 100000
B = 4
NSHIFT = (H // 2, W // 2)

NCHUNK = 16
NWORK = 32
GPW = 25
GSIZE = 128
NP = NWORK * GPW * GSIZE

I0_ALPHA = float(np.i0(ALPHA))
KSCALE0 = KH / (2.0 * math.pi)
KSCALE1 = KW / (2.0 * math.pi)


def _np_kb_ft(om, numpoints, alpha):
    z2 = (np.pi * numpoints * om) ** 2 - alpha**2
    z = np.sqrt(z2.astype(np.complex128))
    return numpoints * np.real(np.sin(z) / z) / np.i0(alpha)


def _np_scaling_coefs(num, grid, numpoints, alpha):
    pos = (np.arange(num, dtype=np.float64) - (num - 1) / 2.0) / grid
    return 1.0 / _np_kb_ft(pos, numpoints, alpha)


_SN = np.outer(
    _np_scaling_coefs(H, KH, J, ALPHA), _np_scaling_coefs(W, KW, J, ALPHA)
).astype(np.float32)

_k = np.arange(KH, dtype=np.float64)[:, None]
_r = np.arange(H, dtype=np.float64)[None, :]
_ANG = 2.0 * np.pi * _k * _r / KH
_C1 = (np.cos(_ANG) / np.sqrt(KH)).astype(np.float32)
_S1 = (np.sin(_ANG) / np.sqrt(KH)).astype(np.float32)


def _dot(a, b, da, db):
    return lax.dot_general(
        a,
        b,
        (((da,), (db,)), ((), ())),
        preferred_element_type=jnp.float32,
        precision=lax.Precision.HIGHEST,
    )


def _dft_body(x_ref, sn_ref, c_ref, s_ref, out_ref):
    c = c_ref[...]
    s = s_ref[...]
    sn = sn_ref[...]
    for b in range(B):
        xs = x_ref[b] * sn
        ac = _dot(c, xs, 1, 0)
        as_ = _dot(s, xs, 1, 0)
        out_ref[b] = _dot(ac, c, 1, 1) - _dot(as_, s, 1, 1)
        out_ref[B + b] = -(_dot(ac, s, 1, 1) + _dot(as_, c, 1, 1))


def _i0(x):
    t = (x * (1.0 / 3.75)) ** 2
    small = 1.0 + t * (
        3.5156229
        + t
        * (3.0899424 + t * (1.2067492 + t * (0.2659732 + t * (0.0360768 + t * 0.0045813))))
    )
    xi = 1.0 / jnp.maximum(x, 3.75)
    u = 3.75 * xi
    big = 0.39894228 + u * (
        0.01328592
        + u
        * (
            0.00225319
            + u
            * (
                -0.00157565
                + u
                * (
                    0.00916281
                    + u * (-0.02057706 + u * (0.02635537 + u * (-0.01647633 + u * 0.00392377)))
                )
            )
        )
    )
    big = big * jnp.exp(x) * jnp.sqrt(xi)
    return jnp.where(x < 3.75, small, big)


def _kb_weight(u):
    arg = 1.0 - (2.0 * u / J) ** 2
    pos = arg > 0
    sq = jnp.sqrt(jnp.where(pos, arg, 1.0))
    w = _i0(ALPHA * jnp.where(pos, sq, 0.0)) * (1.0 / I0_ALPHA)
    return jnp.where(pos, w, 0.0)


def _prep_body(
    u0_ref, u1_ref, w0_ref, w1_ref, rb_ref, hi_ref, hi2_ref, off_ref, pc_ref, ps_ref
):
    u0 = u0_ref[...]
    u1 = u1_ref[...]
    tm0 = u0 * KSCALE0
    tm1 = u1 * KSCALE1
    r0 = jnp.rint(tm0)
    r1 = jnp.rint(tm1)
    f0 = tm0 - r0
    f1 = tm1 - r1
    for j in range(J):
        off = float(j) - (J - 1) / 2.0
        w0_ref[j] = _kb_weight(f0 - off)
        w1_ref[j] = _kb_weight(f1 - off)
        k0 = (r0 + off).astype(jnp.int32)
        rb_ref[j] = jnp.mod(k0, KH) * 32
    c0 = jnp.mod((r1 - 3.0).astype(jnp.int32), KW)
    hi = lax.shift_right_logical(c0, 4)
    hi_ref[...] = hi
    hi2_ref[...] = jnp.bitwise_and(hi + 1, 31)
    off_ref[...] = jnp.bitwise_and(c0, 15) * 8
    phase = u0 * float(NSHIFT[0]) + u1 * float(NSHIFT[1])
    pc_ref[...] = jnp.cos(phase)
    ps_ref[...] = jnp.sin(phase)


def _interp_body(
    table,
    rb,
    hi,
    hi2,
    off0,
    w0,
    w1,
    pc,
    ps,
    out,
    rbg,
    w0g,
    w1g,
    hig,
    hi2g,
    offg,
    pcg,
    psg,
    idx_s,
    buf_s,
    idx_b,
    buf_b,
    outg,
    sem,
    sem_b,
):
    wid = lax.axis_index("s") * 2 + lax.axis_index("c")
    iota16 = lax.iota(jnp.int32, 16)
    zero16 = jnp.zeros((16,), jnp.float32)

    def group(g, carry):
        gbase = (wid * GPW + g) * GSIZE
        pltpu.sync_copy(rb.at[:, pl.ds(gbase, GSIZE)], rbg)
        pltpu.sync_copy(w0.at[:, pl.ds(gbase, GSIZE)], w0g)
        pltpu.sync_copy(w1.at[:, pl.ds(gbase, GSIZE)], w1g)
        pltpu.sync_copy(hi.at[pl.ds(gbase, GSIZE)], hig)
        pltpu.sync_copy(hi2.at[pl.ds(gbase, GSIZE)], hi2g)
        pltpu.sync_copy(off0.at[pl.ds(gbase, GSIZE)], offg)
        pltpu.sync_copy(pc.at[pl.ds(gbase, GSIZE)], pcg)
        pltpu.sync_copy(ps.at[pl.ds(gbase, GSIZE)], psg)

        ie = iota16 * 2

        def build_fire(k, idx_r, buf_r, sm):
            off = k * NCHUNK
            hiv = hig[pl.ds(off, NCHUNK)]
            hi2v = hi2g[pl.ds(off, NCHUNK)]
            for j0 in range(J):
                rbv = rbg[j0, pl.ds(off, NCHUNK)]
                plsc.store_scatter(idx_r.at[j0], [ie], rbv + hiv)
                plsc.store_scatter(idx_r.at[j0], [ie + 1], rbv + hi2v)
            for j0 in range(J):
                pltpu.async_copy(
                    table.at[idx_r.at[j0]], buf_r.at[pl.ds(j0 * 32, 32)], sm
                )

        def drain(idx_r, buf_r, sm):
            for j0 in range(J):
                pltpu.make_async_copy(
                    table.at[idx_r.at[j0]], buf_r.at[pl.ds(j0 * 32, 32)], sm
                ).wait()

        def compute(k, buf_r):
            off = k * NCHUNK
            offv = offg[pl.ds(off, NCHUNK)]
            w0l = [w0g[j, pl.ds(off, NCHUNK)] for j in range(J)]
            w1l = [w1g[j, pl.ds(off, NCHUNK)] for j in range(J)]
            sjl = []
            cjl = []
            for j1 in range(J):
                tj = offv + (j1 * 8)
                sjl.append(lax.shift_right_logical(tj, 7))
                cjl.append(jnp.bitwise_and(tj, 127))
            acc = [zero16 for _ in range(8)]
            for j0 in range(J):
                rowb = ie + (j0 * 32)
                for j1 in range(J):
                    wt = w0l[j0] * w1l[j1]
                    rowj = rowb + sjl[j1]
                    cj = cjl[j1]
                    for u in range(8):
                        v = plsc.load_gather(buf_r, [rowj, cj + u])
                        acc[u] = acc[u] + wt * v
            pcl = pcg[pl.ds(off, NCHUNK)]
            psl = psg[pl.ds(off, NCHUNK)]
            for c in range(4):
                outg[c, pl.ds(off, NCHUNK)] = acc[c] * pcl - acc[4 + c] * psl
                outg[4 + c, pl.ds(off, NCHUNK)] = acc[c] * psl + acc[4 + c] * pcl

        nchunks = GSIZE // NCHUNK
        build_fire(0, idx_s, buf_s, sem)

        def pair(t, carry2):
            a = t * 2
            build_fire(a + 1, idx_b, buf_b, sem_b)
            drain(idx_s, buf_s, sem)
            compute(a, buf_s)
            build_fire(a + 2, idx_s, buf_s, sem)
            drain(idx_b, buf_b, sem_b)
            compute(a + 1, buf_b)
            return carry2

        lax.fori_loop(0, nchunks // 2 - 1, pair, 0)
        build_fire(nchunks - 1, idx_b, buf_b, sem_b)
        drain(idx_s, buf_s, sem)
        compute(nchunks - 2, buf_s)
        drain(idx_b, buf_b, sem_b)
        compute(nchunks - 1, buf_b)
        pltpu.sync_copy(outg, out.at[:, pl.ds(gbase, GSIZE)])
        return carry

    lax.fori_loop(0, GPW, group, 0)


def kernel(x, uv):
    xb = x.reshape(B, H, W)
    up = jnp.pad(uv, ((0, 0), (0, NP - N)))
    u0 = up[0].reshape(NP // 128, 128)
    u1 = up[1].reshape(NP // 128, 128)

    sn = jnp.asarray(_SN)
    c1 = jnp.asarray(_C1)
    s1 = jnp.asarray(_S1)

    stacked = pl.pallas_call(
        _dft_body,
        out_shape=jax.ShapeDtypeStruct((2 * B, KH, KW), jnp.float32),
    )(xb, sn, c1, s1)
    table = stacked.transpose(1, 2, 0).reshape(KH * 32, 16 * 2 * B)

    pshape = jax.ShapeDtypeStruct((J, NP // 128, 128), jnp.float32)
    ishape = jax.ShapeDtypeStruct((J, NP // 128, 128), jnp.int32)
    vshape = jax.ShapeDtypeStruct((NP // 128, 128), jnp.float32)
    sshape = jax.ShapeDtypeStruct((NP // 128, 128), jnp.int32)
    w0, w1, rb, hi, hi2, off0, pc, ps = pl.pallas_call(
        _prep_body,
        out_shape=(pshape, pshape, ishape, sshape, sshape, sshape, vshape, vshape),
    )(u0, u1)

    mesh = plsc.VectorSubcoreMesh(
        core_axis_name="c", subcore_axis_name="s", num_cores=2, num_subcores=16
    )
    interp = pl.kernel(
        _interp_body,
        out_type=jax.ShapeDtypeStruct((8, NP), jnp.float32),
        mesh=mesh,
        scratch_types=[
            pltpu.VMEM((J, GSIZE), jnp.int32),
            pltpu.VMEM((J, GSIZE), jnp.float32),
            pltpu.VMEM((J, GSIZE), jnp.float32),
            pltpu.VMEM((GSIZE,), jnp.int32),
            pltpu.VMEM((GSIZE,), jnp.int32),
            pltpu.VMEM((GSIZE,), jnp.int32),
            pltpu.VMEM((GSIZE,), jnp.float32),
            pltpu.VMEM((GSIZE,), jnp.float32),
            pltpu.VMEM((J, 32), jnp.int32),
            pltpu.VMEM((J * 32, 128), jnp.float32),
            pltpu.VMEM((J, 32), jnp.int32),
            pltpu.VMEM((J * 32, 128), jnp.float32),
            pltpu.VMEM((8, GSIZE), jnp.float32),
            pltpu.SemaphoreType.DMA,
            pltpu.SemaphoreType.DMA,
        ],
        compiler_params=pltpu.CompilerParams(needs_layout_passes=False),
    )
    res = interp(
        table,
        rb.reshape(J, NP),
        hi.reshape(NP),
        hi2.reshape(NP),
        off0.reshape(NP),
        w0.reshape(J, NP),
        w1.reshape(J, NP),
        pc.reshape(NP),
        ps.reshape(NP),
    )
    kdata = res[0:B, :N] + 1j * res[B : 2 * B, :N]
    return kdata.astype(jnp.complex64)[:, None, :]


# speedup vs baseline: 152.8833x; 1.1039x over previous
import functools
import math

import jax
import jax.numpy as jnp
import numpy as np
from jax import lax
from jax.experimental import pallas as pl
from jax.experimental.pallas import tpu as pltpu
from jax.experimental.pallas import tpu_sc as plsc

H = 256
W = 256
KH = 512
KW = 512
J = 7
ALPHA = 2.34 * J
N = 100000
B = 4
NSHIFT = (H // 2, W // 2)

NCHUNK = 16
NWORK = 32
GPW = 25
GSIZE = 128
NP = NWORK * GPW * GSIZE

I0_ALPHA = float(np.i0(ALPHA))
KSCALE0 = KH / (2.0 * math.pi)
KSCALE1 = KW / (2.0 * math.pi)


def _np_kb_ft(om, numpoints, alpha):
    z2 = (np.pi * numpoints * om) ** 2 - alpha**2
    z = np.sqrt(z2.astype(np.complex128))
    return numpoints * np.real(np.sin(z) / z) / np.i0(alpha)


def _np_scaling_coefs(num, grid, numpoints, alpha):
    pos = (np.arange(num, dtype=np.float64) - (num - 1) / 2.0) / grid
    return 1.0 / _np_kb_ft(pos, numpoints, alpha)


_SN = np.outer(
    _np_scaling_coefs(H, KH, J, ALPHA), _np_scaling_coefs(W, KW, J, ALPHA)
).astype(np.float32)

_k = np.arange(KH, dtype=np.float64)[:, None]
_r = np.arange(H, dtype=np.float64)[None, :]
_ANG = 2.0 * np.pi * _k * _r / KH
_C1 = (np.cos(_ANG) / np.sqrt(KH)).astype(np.float32)
_S1 = (np.sin(_ANG) / np.sqrt(KH)).astype(np.float32)


def _dot(a, b, da, db):
    return lax.dot_general(
        a,
        b,
        (((da,), (db,)), ((), ())),
        preferred_element_type=jnp.float32,
        precision=lax.Precision.HIGHEST,
    )


def _dft_body(x_ref, sn_ref, c_ref, s_ref, out_ref):
    c = c_ref[...]
    s = s_ref[...]
    sn = sn_ref[...]
    for b in range(B):
        xs = x_ref[b] * sn
        ac = _dot(c, xs, 1, 0)
        as_ = _dot(s, xs, 1, 0)
        out_ref[b] = _dot(ac, c, 1, 1) - _dot(as_, s, 1, 1)
        out_ref[B + b] = -(_dot(ac, s, 1, 1) + _dot(as_, c, 1, 1))


def _i0(x):
    t = (x * (1.0 / 3.75)) ** 2
    small = 1.0 + t * (
        3.5156229
        + t
        * (3.0899424 + t * (1.2067492 + t * (0.2659732 + t * (0.0360768 + t * 0.0045813))))
    )
    xi = 1.0 / jnp.maximum(x, 3.75)
    u = 3.75 * xi
    big = 0.39894228 + u * (
        0.01328592
        + u
        * (
            0.00225319
            + u
            * (
                -0.00157565
                + u
                * (
                    0.00916281
                    + u * (-0.02057706 + u * (0.02635537 + u * (-0.01647633 + u * 0.00392377)))
                )
            )
        )
    )
    big = big * jnp.exp(x) * jnp.sqrt(xi)
    return jnp.where(x < 3.75, small, big)


def _kb_weight(u):
    arg = 1.0 - (2.0 * u / J) ** 2
    pos = arg > 0
    sq = jnp.sqrt(jnp.where(pos, arg, 1.0))
    w = _i0(ALPHA * jnp.where(pos, sq, 0.0)) * (1.0 / I0_ALPHA)
    return jnp.where(pos, w, 0.0)


def _prep_body(
    u0_ref, u1_ref, w0_ref, w1_ref, s1_ref, s2_ref, off_ref, pc_ref, ps_ref
):
    u0 = u0_ref[...]
    u1 = u1_ref[...]
    tm0 = u0 * KSCALE0
    tm1 = u1 * KSCALE1
    r0 = jnp.rint(tm0)
    r1 = jnp.rint(tm1)
    f0 = tm0 - r0
    f1 = tm1 - r1
    for j in range(J):
        off = float(j) - (J - 1) / 2.0
        w0_ref[j] = _kb_weight(f0 - off)
        w1_ref[j] = _kb_weight(f1 - off)
    r0m = jnp.mod(r0.astype(jnp.int32), KH)
    c0 = jnp.mod((r1 - 3.0).astype(jnp.int32), KW)
    hi = lax.shift_right_logical(c0, 4)
    hi2 = jnp.bitwise_and(hi + 1, 31)
    s1_ref[...] = hi * (KH + 6) + r0m
    s2_ref[...] = hi2 * (KH + 6) + r0m
    off_ref[...] = jnp.bitwise_and(c0, 15) * 8
    phase = u0 * float(NSHIFT[0]) + u1 * float(NSHIFT[1])
    pc_ref[...] = jnp.cos(phase)
    ps_ref[...] = jnp.sin(phase)


def _interp_body(
    table,
    s1,
    s2,
    off0,
    w0,
    w1,
    pc,
    ps,
    out,
    w0g,
    w1g,
    s1g,
    s2g,
    offg,
    pcg,
    psg,
    idx_s,
    buf_s,
    idx_b,
    buf_b,
    outg,
    sem,
    sem_b,
):
    wid = lax.axis_index("s") * 2 + lax.axis_index("c")
    iota16 = lax.iota(jnp.int32, 16)
    zero16 = jnp.zeros((16,), jnp.float32)

    def group(g, carry):
        gbase = (wid * GPW + g) * GSIZE
        pltpu.sync_copy(w0.at[:, pl.ds(gbase, GSIZE)], w0g)
        pltpu.sync_copy(w1.at[:, pl.ds(gbase, GSIZE)], w1g)
        pltpu.sync_copy(s1.at[pl.ds(gbase, GSIZE)], s1g)
        pltpu.sync_copy(s2.at[pl.ds(gbase, GSIZE)], s2g)
        pltpu.sync_copy(off0.at[pl.ds(gbase, GSIZE)], offg)
        pltpu.sync_copy(pc.at[pl.ds(gbase, GSIZE)], pcg)
        pltpu.sync_copy(ps.at[pl.ds(gbase, GSIZE)], psg)

        ie = iota16 * 2

        def build_fire(k, idx_r, buf_r, sm):
            off = k * NCHUNK
            s1v = s1g[pl.ds(off, NCHUNK)]
            s2v = s2g[pl.ds(off, NCHUNK)]
            for j0 in range(J):
                plsc.store_scatter(idx_r.at[j0], [ie], s1v + j0)
                plsc.store_scatter(idx_r.at[j0], [ie + 1], s2v + j0)
            for j0 in range(J):
                pltpu.async_copy(
                    table.at[idx_r.at[j0]], buf_r.at[pl.ds(j0 * 32, 32)], sm
                )

        def drain(idx_r, buf_r, sm):
            for j0 in range(J):
                pltpu.make_async_copy(
                    table.at[idx_r.at[j0]], buf_r.at[pl.ds(j0 * 32, 32)], sm
                ).wait()

        def compute(k, buf_r):
            off = k * NCHUNK
            offv = offg[pl.ds(off, NCHUNK)]
            w0l = [w0g[j, pl.ds(off, NCHUNK)] for j in range(J)]
            w1l = [w1g[j, pl.ds(off, NCHUNK)] for j in range(J)]
            sjl = []
            cjl = []
            for j1 in range(J):
                tj = offv + (j1 * 8)
                sjl.append(lax.shift_right_logical(tj, 7))
                cjl.append(jnp.bitwise_and(tj, 127))
            acc = [zero16 for _ in range(8)]
            for j0 in range(J):
                rowb = ie + (j0 * 32)
                for j1 in range(J):
                    wt = w0l[j0] * w1l[j1]
                    rowj = rowb + sjl[j1]
                    cj = cjl[j1]
                    for u in range(8):
                        v = plsc.load_gather(buf_r, [rowj, cj + u])
                        acc[u] = acc[u] + wt * v
            pcl = pcg[pl.ds(off, NCHUNK)]
            psl = psg[pl.ds(off, NCHUNK)]
            for c in range(4):
                outg[c, pl.ds(off, NCHUNK)] = acc[c] * pcl - acc[4 + c] * psl
                outg[4 + c, pl.ds(off, NCHUNK)] = acc[c] * psl + acc[4 + c] * pcl

        nchunks = GSIZE // NCHUNK
        build_fire(0, idx_s, buf_s, sem)

        def pair(t, carry2):
            a = t * 2
            build_fire(a + 1, idx_b, buf_b, sem_b)
            drain(idx_s, buf_s, sem)
            compute(a, buf_s)
            build_fire(a + 2, idx_s, buf_s, sem)
            drain(idx_b, buf_b, sem_b)
            compute(a + 1, buf_b)
            return carry2

        lax.fori_loop(0, nchunks // 2 - 1, pair, 0)
        build_fire(nchunks - 1, idx_b, buf_b, sem_b)
        drain(idx_s, buf_s, sem)
        compute(nchunks - 2, buf_s)
        drain(idx_b, buf_b, sem_b)
        compute(nchunks - 1, buf_b)
        pltpu.sync_copy(outg, out.at[:, pl.ds(gbase, GSIZE)])
        return carry

    lax.fori_loop(0, GPW, group, 0)


def kernel(x, uv):
    xb = x.reshape(B, H, W)
    up = jnp.pad(uv, ((0, 0), (0, NP - N)))
    u0 = up[0].reshape(NP // 128, 128)
    u1 = up[1].reshape(NP // 128, 128)

    sn = jnp.asarray(_SN)
    c1 = jnp.asarray(_C1)
    s1 = jnp.asarray(_S1)

    stacked = pl.pallas_call(
        _dft_body,
        out_shape=jax.ShapeDtypeStruct((2 * B, KH, KW), jnp.float32),
    )(xb, sn, c1, s1)
    tcb = stacked.transpose(1, 2, 0).reshape(KH, 32, 128).transpose(1, 0, 2)
    tcb = jnp.concatenate([tcb[:, -3:], tcb, tcb[:, :3]], axis=1)
    table = tcb.reshape(32 * (KH + 6), 128)

    pshape = jax.ShapeDtypeStruct((J, NP // 128, 128), jnp.float32)
    vshape = jax.ShapeDtypeStruct((NP // 128, 128), jnp.float32)
    sshape = jax.ShapeDtypeStruct((NP // 128, 128), jnp.int32)
    w0, w1, s1p, s2p, off0, pc, ps = pl.pallas_call(
        _prep_body,
        out_shape=(pshape, pshape, sshape, sshape, sshape, vshape, vshape),
    )(u0, u1)

    mesh = plsc.VectorSubcoreMesh(
        core_axis_name="c", subcore_axis_name="s", num_cores=2, num_subcores=16
    )
    interp = pl.kernel(
        _interp_body,
        out_type=jax.ShapeDtypeStruct((8, NP), jnp.float32),
        mesh=mesh,
        scratch_types=[
            pltpu.VMEM((J, GSIZE), jnp.float32),
            pltpu.VMEM((J, GSIZE), jnp.float32),
            pltpu.VMEM((GSIZE,), jnp.int32),
            pltpu.VMEM((GSIZE,), jnp.int32),
            pltpu.VMEM((GSIZE,), jnp.int32),
            pltpu.VMEM((GSIZE,), jnp.float32),
            pltpu.VMEM((GSIZE,), jnp.float32),
            pltpu.VMEM((J, 32), jnp.int32),
            pltpu.VMEM((J * 32, 128), jnp.float32),
            pltpu.VMEM((J, 32), jnp.int32),
            pltpu.VMEM((J * 32, 128), jnp.float32),
            pltpu.VMEM((8, GSIZE), jnp.float32),
            pltpu.SemaphoreType.DMA,
            pltpu.SemaphoreType.DMA,
        ],
        compiler_params=pltpu.CompilerParams(needs_layout_passes=False),
    )
    res = interp(
        table,
        s1p.reshape(NP),
        s2p.reshape(NP),
        off0.reshape(NP),
        w0.reshape(J, NP),
        w1.reshape(J, NP),
        pc.reshape(NP),
        ps.reshape(NP),
    )
    kdata = res[0:B, :N] + 1j * res[B : 2 * B, :N]
    return kdata.astype(jnp.complex64)[:, None, :]
